# baseline (speedup 1.0000x reference)
import functools

import jax
import jax.numpy as jnp
from jax import lax
from jax.experimental import pallas as pl
from jax.experimental.pallas import tpu as pltpu
from jax.experimental.pallas import tpu_sc as plsc

B = 16
Q = 5000
C = 92
N = B * Q
THRESH = 0.7


BLK = 2000
NBLK = N // BLK


def _tc_body(lg_ref, sc_ref, lb_ref):
    x = lg_ref[...]
    cidx = lax.broadcasted_iota(jnp.int32, (BLK, C), 1)
    elig = cidx < C - 1
    neg_inf = jnp.float32(-jnp.inf)
    m_all = jnp.max(x, axis=1)
    x91 = jnp.where(elig, x, neg_inf)
    m91 = jnp.max(x91, axis=1)
    lbl = jnp.min(jnp.where(x91 == m91[:, None], cidx, C), axis=1)
    s = jnp.sum(jnp.exp(x - m_all[:, None]), axis=1)
    score = jnp.exp(m91 - m_all) / s
    keep = score > THRESH
    sc_ref[0, 0, :] = jnp.where(keep, score, 0.0)
    lb_ref[0, 0, :] = jnp.where(keep, lbl, 0)


_tc_scores = pl.pallas_call(
    _tc_body,
    grid=(NBLK,),
    in_specs=[pl.BlockSpec((BLK, C), lambda i: (i, 0))],
    out_specs=[
        pl.BlockSpec((1, 1, BLK), lambda i: (i, 0, 0)),
        pl.BlockSpec((1, 1, BLK), lambda i: (i, 0, 0)),
    ],
    out_shape=[
        jax.ShapeDtypeStruct((NBLK, 1, BLK), jnp.float32),
        jax.ShapeDtypeStruct((NBLK, 1, BLK), jnp.int32),
    ],
)


EPW = N * 4 // 32
BPW = N // 32


def _sc_body(boxes, scores, scale, boxes_o, bbuf, obuf, sbuf, scbuf):
    w = lax.axis_index("s") * 2 + lax.axis_index("c")
    ebase = w * EPW
    bbase = w * BPW
    align = (bbase // 8) * 8
    r = bbase - align

    pltpu.sync_copy(scale, scbuf)
    pltpu.sync_copy(boxes.at[pl.ds(ebase, EPW)], bbuf)
    pltpu.sync_copy(scores.at[pl.ds(align, BPW + 8)], sbuf)

    lane = lax.iota(jnp.int32, 16)
    lane_div4 = lane // 4
    lane_mod4 = lane - 4 * lane_div4

    def step(v, carry):
        le = v * 16
        bl = le // 4 + lane_div4
        batch = (bbase + bl) // Q
        swh = plsc.load_gather(scbuf, [batch * 4 + lane_mod4])
        scv = plsc.load_gather(sbuf, [bl + r])
        bx = bbuf[pl.ds(le, 16)]
        obuf[pl.ds(le, 16)] = jnp.where(scv > THRESH, bx * swh, 0.0)
        return carry

    lax.fori_loop(0, EPW // 16, step, 0)
    pltpu.sync_copy(obuf, boxes_o.at[pl.ds(ebase, EPW)])


_sc_boxes = pl.kernel(
    _sc_body,
    out_type=jax.ShapeDtypeStruct((N * 4,), jnp.float32),
    mesh=plsc.VectorSubcoreMesh(core_axis_name="c", subcore_axis_name="s"),
    scratch_types=[
        pltpu.VMEM((EPW,), jnp.float32),
        pltpu.VMEM((EPW,), jnp.float32),
        pltpu.VMEM((BPW + 8,), jnp.float32),
        pltpu.VMEM((64,), jnp.float32),
    ],
    compiler_params=pltpu.CompilerParams(needs_layout_passes=False),
)


@jax.jit
def kernel(pred_logits, pred_boxes, target_sizes):
    lg = pred_logits.reshape(N, C)
    scores_b, labels_b = _tc_scores(lg)
    scores_f = scores_b.reshape(N)
    labels_f = labels_b.reshape(N)

    ts = target_sizes.astype(jnp.float32)
    img_h = ts[:, 0]
    img_w = ts[:, 1]
    scale = jnp.stack([img_w, img_h, img_w, img_h], axis=1).reshape(-1)

    boxes_f = _sc_boxes(pred_boxes.reshape(-1), scores_f, scale)

    scores = scores_f.reshape(B, Q)
    labels = labels_f.reshape(B, Q)
    boxes = boxes_f.reshape(B, Q, 4)
    keep = scores > THRESH
    return scores, labels, boxes, keep


# speedup vs baseline: 4.9178x; 4.9178x over previous
import jax
import jax.numpy as jnp
from jax import lax
from jax.experimental import pallas as pl
from jax.experimental.pallas import tpu as pltpu
from jax.experimental.pallas import tpu_sc as plsc

B = 16
Q = 5000
C = 92
N = B * Q
THRESH = 0.7


P = 4
NSTEP = C // P


def _tc_body(x_ref, sc_ref, lb_ref, m91r, mAr, sr, lblr):
    i = pl.program_id(0)

    @pl.when(i == 0)
    def _():
        m = x_ref[0]
        lbl = jnp.zeros((B, Q), jnp.int32)
        for k in range(1, P):
            xk = x_ref[k]
            gt = xk > m
            m = jnp.where(gt, xk, m)
            lbl = jnp.where(gt, k, lbl)
        m91r[...] = m
        lblr[...] = lbl

    @pl.when((i > 0) & (i < NSTEP - 1))
    def _():
        m = m91r[...]
        lbl = lblr[...]
        for k in range(P):
            xk = x_ref[k]
            gt = xk > m
            m = jnp.where(gt, xk, m)
            lbl = jnp.where(gt, P * i + k, lbl)
        m91r[...] = m
        lblr[...] = lbl

    @pl.when(i == NSTEP - 1)
    def _():
        m = m91r[...]
        lbl = lblr[...]
        for k in range(P - 1):
            xk = x_ref[k]
            gt = xk > m
            m = jnp.where(gt, xk, m)
            lbl = jnp.where(gt, P * (NSTEP - 1) + k, lbl)
        m91r[...] = m
        lblr[...] = lbl
        mAr[...] = jnp.maximum(m, x_ref[P - 1])
        sr[...] = jnp.zeros((B, Q), jnp.float32)

    @pl.when(i >= NSTEP)
    def _():
        mA = mAr[...]
        acc = sr[...]
        for k in range(P):
            acc = acc + jnp.exp(x_ref[k] - mA)
        sr[...] = acc

    @pl.when(i == 2 * NSTEP - 1)
    def _():
        score = jnp.exp(m91r[...] - mAr[...]) / sr[...]
        keep = score > THRESH
        sc_ref[...] = jnp.where(keep, score, 0.0)
        lb_ref[...] = jnp.where(keep, lblr[...], 0)


_tc_scores = pl.pallas_call(
    _tc_body,
    grid=(2 * NSTEP,),
    in_specs=[
        pl.BlockSpec((P, B, Q), lambda i: (jnp.where(i < NSTEP, i, i - NSTEP), 0, 0))
    ],
    out_specs=[
        pl.BlockSpec((B, Q), lambda i: (0, 0)),
        pl.BlockSpec((B, Q), lambda i: (0, 0)),
    ],
    out_shape=[
        jax.ShapeDtypeStruct((B, Q), jnp.float32),
        jax.ShapeDtypeStruct((B, Q), jnp.int32),
    ],
    scratch_shapes=[
        pltpu.VMEM((B, Q), jnp.float32),
        pltpu.VMEM((B, Q), jnp.float32),
        pltpu.VMEM((B, Q), jnp.float32),
        pltpu.VMEM((B, Q), jnp.int32),
    ],
)


EPW = 2 * Q
VA = (Q - 16) // 16
VB = (Q - 8) // 16


def _sc_body(boxes, scores, scale, boxes_o, bbuf, obuf, sbuf, scbuf):
    w = lax.axis_index("s") * 2 + lax.axis_index("c")
    ebase = w * EPW
    b = w // 2
    r0 = 2 * w

    pltpu.sync_copy(scale, scbuf)
    pltpu.sync_copy(boxes.at[pl.ds(ebase, EPW)], bbuf)
    pltpu.sync_copy(scores.at[pl.ds(b * Q, Q)], sbuf)

    lane = lax.iota(jnp.int32, 16)
    s0 = plsc.load_gather(scbuf, [jnp.zeros((16,), jnp.int32) + r0])
    s1 = plsc.load_gather(scbuf, [jnp.zeros((16,), jnp.int32) + (r0 + 1)])

    def step_a(v, carry):
        le = v * 16
        bx = bbuf[pl.ds(le, 16)]
        sc = sbuf[pl.ds(le, 16)]
        obuf[pl.ds(le, 16)] = jnp.where(sc > THRESH, bx * s0, 0.0)
        return carry

    lax.fori_loop(0, VA + 1, step_a, 0)

    qb = jnp.where(lane < 8, (Q - 8) + lane, lane - 8)
    scb = plsc.load_gather(sbuf, [qb])
    smix = jnp.where(lane < 8, s0, s1)
    bxb = bbuf[pl.ds(Q - 8, 16)]
    obuf[pl.ds(Q - 8, 16)] = jnp.where(scb > THRESH, bxb * smix, 0.0)

    def step_b(v, carry):
        le = Q + 8 + v * 16
        bx = bbuf[pl.ds(le, 16)]
        sc = sbuf[pl.ds(le - Q, 16)]
        obuf[pl.ds(le, 16)] = jnp.where(sc > THRESH, bx * s1, 0.0)
        return carry

    lax.fori_loop(0, VB, step_b, 0)

    pltpu.sync_copy(obuf, boxes_o.at[pl.ds(ebase, EPW)])


_sc_boxes = pl.kernel(
    _sc_body,
    out_type=jax.ShapeDtypeStruct((N * 4,), jnp.float32),
    mesh=plsc.VectorSubcoreMesh(core_axis_name="c", subcore_axis_name="s"),
    scratch_types=[
        pltpu.VMEM((EPW,), jnp.float32),
        pltpu.VMEM((EPW,), jnp.float32),
        pltpu.VMEM((Q,), jnp.float32),
        pltpu.VMEM((64,), jnp.float32),
    ],
    compiler_params=pltpu.CompilerParams(needs_layout_passes=False),
)


@jax.jit
def kernel(pred_logits, pred_boxes, target_sizes):
    lgT = jnp.transpose(pred_logits, (2, 0, 1))
    scores2d, labels2d = _tc_scores(lgT)

    ts = target_sizes.astype(jnp.float32)
    img_h = ts[:, 0]
    img_w = ts[:, 1]
    scale = jnp.stack([img_w, img_h, img_w, img_h], axis=1).reshape(-1)

    bxt = jnp.transpose(pred_boxes, (0, 2, 1)).reshape(-1)
    boxes_t = _sc_boxes(bxt, scores2d.reshape(N), scale)
    boxes = jnp.transpose(boxes_t.reshape(B, 4, Q), (0, 2, 1))

    keep = scores2d > THRESH
    return scores2d, labels2d, boxes, keep


# speedup vs baseline: 6.3153x; 1.2842x over previous
import jax
import jax.numpy as jnp
from jax import lax
from jax.experimental import pallas as pl
from jax.experimental.pallas import tpu as pltpu
from jax.experimental.pallas import tpu_sc as plsc

B = 16
Q = 5000
C = 92
N = B * Q
THRESH = 0.7


P = 4
NSTEP = C // P


def _tc_body(x_ref, sc_ref, lb_ref, m91r, mAr, sr, lblr):
    i = pl.program_id(0)
    xs = [x_ref[k] for k in range(P)]
    cmax = jnp.maximum(jnp.maximum(xs[0], xs[1]), jnp.maximum(xs[2], xs[3]))

    @pl.when(i == 0)
    def _():
        m = xs[0]
        lbl = jnp.zeros((B, Q), jnp.int32)
        for k in range(1, P):
            gt = xs[k] > m
            m = jnp.where(gt, xs[k], m)
            lbl = jnp.where(gt, k, lbl)
        m91r[...] = m
        lblr[...] = lbl
        mAr[...] = cmax
        acc = jnp.exp(xs[0] - cmax)
        for k in range(1, P):
            acc = acc + jnp.exp(xs[k] - cmax)
        sr[...] = acc

    @pl.when((i > 0) & (i < NSTEP - 1))
    def _():
        m = m91r[...]
        lbl = lblr[...]
        for k in range(P):
            gt = xs[k] > m
            m = jnp.where(gt, xs[k], m)
            lbl = jnp.where(gt, P * i + k, lbl)
        m91r[...] = m
        lblr[...] = lbl
        mo = mAr[...]
        mn = jnp.maximum(mo, cmax)
        acc = sr[...] * jnp.exp(mo - mn)
        for k in range(P):
            acc = acc + jnp.exp(xs[k] - mn)
        sr[...] = acc
        mAr[...] = mn

    @pl.when(i == NSTEP - 1)
    def _():
        m = m91r[...]
        lbl = lblr[...]
        for k in range(P - 1):
            gt = xs[k] > m
            m = jnp.where(gt, xs[k], m)
            lbl = jnp.where(gt, P * (NSTEP - 1) + k, lbl)
        mo = mAr[...]
        mn = jnp.maximum(mo, cmax)
        acc = sr[...] * jnp.exp(mo - mn)
        for k in range(P):
            acc = acc + jnp.exp(xs[k] - mn)
        score = jnp.exp(m - mn) / acc
        keep = score > THRESH
        sc_ref[...] = jnp.where(keep, score, 0.0)
        lb_ref[...] = jnp.where(keep, lbl, 0)


_tc_scores = pl.pallas_call(
    _tc_body,
    grid=(NSTEP,),
    in_specs=[
        pl.BlockSpec((P, B, Q), lambda i: (i, 0, 0))
    ],
    out_specs=[
        pl.BlockSpec((B, Q), lambda i: (0, 0)),
        pl.BlockSpec((B, Q), lambda i: (0, 0)),
    ],
    out_shape=[
        jax.ShapeDtypeStruct((B, Q), jnp.float32),
        jax.ShapeDtypeStruct((B, Q), jnp.int32),
    ],
    scratch_shapes=[
        pltpu.VMEM((B, Q), jnp.float32),
        pltpu.VMEM((B, Q), jnp.float32),
        pltpu.VMEM((B, Q), jnp.float32),
        pltpu.VMEM((B, Q), jnp.int32),
    ],
)


EPW = 2 * Q
VA = (Q - 16) // 16
VB = (Q - 8) // 16


def _sc_body(boxes, scores, scale, boxes_o, bbuf, obuf, sbuf, scbuf):
    w = lax.axis_index("s") * 2 + lax.axis_index("c")
    ebase = w * EPW
    b = w // 2
    r0 = 2 * w

    pltpu.sync_copy(scale, scbuf)
    pltpu.sync_copy(boxes.at[pl.ds(ebase, EPW)], bbuf)
    pltpu.sync_copy(scores.at[pl.ds(b * Q, Q)], sbuf)

    lane = lax.iota(jnp.int32, 16)
    s0 = plsc.load_gather(scbuf, [jnp.zeros((16,), jnp.int32) + r0])
    s1 = plsc.load_gather(scbuf, [jnp.zeros((16,), jnp.int32) + (r0 + 1)])

    def step_a(v, carry):
        le = v * 16
        bx = bbuf[pl.ds(le, 16)]
        sc = sbuf[pl.ds(le, 16)]
        obuf[pl.ds(le, 16)] = jnp.where(sc > THRESH, bx * s0, 0.0)
        return carry

    lax.fori_loop(0, VA + 1, step_a, 0)

    qb = jnp.where(lane < 8, (Q - 8) + lane, lane - 8)
    scb = plsc.load_gather(sbuf, [qb])
    smix = jnp.where(lane < 8, s0, s1)
    bxb = bbuf[pl.ds(Q - 8, 16)]
    obuf[pl.ds(Q - 8, 16)] = jnp.where(scb > THRESH, bxb * smix, 0.0)

    def step_b(v, carry):
        le = Q + 8 + v * 16
        bx = bbuf[pl.ds(le, 16)]
        sc = sbuf[pl.ds(le - Q, 16)]
        obuf[pl.ds(le, 16)] = jnp.where(sc > THRESH, bx * s1, 0.0)
        return carry

    lax.fori_loop(0, VB, step_b, 0)

    pltpu.sync_copy(obuf, boxes_o.at[pl.ds(ebase, EPW)])


_sc_boxes = pl.kernel(
    _sc_body,
    out_type=jax.ShapeDtypeStruct((N * 4,), jnp.float32),
    mesh=plsc.VectorSubcoreMesh(core_axis_name="c", subcore_axis_name="s"),
    scratch_types=[
        pltpu.VMEM((EPW,), jnp.float32),
        pltpu.VMEM((EPW,), jnp.float32),
        pltpu.VMEM((Q,), jnp.float32),
        pltpu.VMEM((64,), jnp.float32),
    ],
    compiler_params=pltpu.CompilerParams(needs_layout_passes=False),
)


@jax.jit
def kernel(pred_logits, pred_boxes, target_sizes):
    lgT = jnp.transpose(pred_logits, (2, 0, 1))
    scores2d, labels2d = _tc_scores(lgT)

    ts = target_sizes.astype(jnp.float32)
    img_h = ts[:, 0]
    img_w = ts[:, 1]
    scale = jnp.stack([img_w, img_h, img_w, img_h], axis=1).reshape(-1)

    bxt = jnp.transpose(pred_boxes, (0, 2, 1)).reshape(-1)
    boxes_t = _sc_boxes(bxt, scores2d.reshape(N), scale)
    boxes = jnp.transpose(boxes_t.reshape(B, 4, Q), (0, 2, 1))

    keep = scores2d > THRESH
    return scores2d, labels2d, boxes, keep


# speedup vs baseline: 6.5200x; 1.0324x over previous
import jax
import jax.numpy as jnp
from jax import lax
from jax.experimental import pallas as pl
from jax.experimental.pallas import tpu as pltpu
from jax.experimental.pallas import tpu_sc as plsc

B = 16
Q = 5000
C = 92
N = B * Q
THRESH = 0.7


P = 4
NSTEP = C // P


def _tc_body(x_ref, sc_ref, lb_ref, m91r, mAr, sr, lblr):
    i = pl.program_id(0)
    xs = [x_ref[k] for k in range(P)]
    cmax = jnp.maximum(jnp.maximum(xs[0], xs[1]), jnp.maximum(xs[2], xs[3]))

    @pl.when(i == 0)
    def _():
        m = xs[0]
        lbl = jnp.zeros((B, Q), jnp.int32)
        for k in range(1, P):
            gt = xs[k] > m
            m = jnp.where(gt, xs[k], m)
            lbl = jnp.where(gt, k, lbl)
        m91r[...] = m
        lblr[...] = lbl
        mAr[...] = cmax
        acc = jnp.exp(xs[0] - cmax)
        for k in range(1, P):
            acc = acc + jnp.exp(xs[k] - cmax)
        sr[...] = acc

    @pl.when((i > 0) & (i < NSTEP - 1))
    def _():
        m = m91r[...]
        lbl = lblr[...]
        for k in range(P):
            gt = xs[k] > m
            m = jnp.where(gt, xs[k], m)
            lbl = jnp.where(gt, P * i + k, lbl)
        m91r[...] = m
        lblr[...] = lbl
        mo = mAr[...]
        mn = jnp.maximum(mo, cmax)
        acc = sr[...] * jnp.exp(mo - mn)
        for k in range(P):
            acc = acc + jnp.exp(xs[k] - mn)
        sr[...] = acc
        mAr[...] = mn

    @pl.when(i == NSTEP - 1)
    def _():
        m = m91r[...]
        lbl = lblr[...]
        for k in range(P - 1):
            gt = xs[k] > m
            m = jnp.where(gt, xs[k], m)
            lbl = jnp.where(gt, P * (NSTEP - 1) + k, lbl)
        mo = mAr[...]
        mn = jnp.maximum(mo, cmax)
        acc = sr[...] * jnp.exp(mo - mn)
        for k in range(P):
            acc = acc + jnp.exp(xs[k] - mn)
        score = jnp.exp(m - mn) / acc
        keep = score > THRESH
        sc_ref[...] = jnp.where(keep, score, 0.0)
        lb_ref[...] = jnp.where(keep, lbl, 0)


_tc_scores = pl.pallas_call(
    _tc_body,
    grid=(NSTEP,),
    in_specs=[
        pl.BlockSpec((P, B, Q), lambda i: (i, 0, 0))
    ],
    out_specs=[
        pl.BlockSpec((B, Q), lambda i: (0, 0)),
        pl.BlockSpec((B, Q), lambda i: (0, 0)),
    ],
    out_shape=[
        jax.ShapeDtypeStruct((B, Q), jnp.float32),
        jax.ShapeDtypeStruct((B, Q), jnp.int32),
    ],
    scratch_shapes=[
        pltpu.VMEM((B, Q), jnp.float32),
        pltpu.VMEM((B, Q), jnp.float32),
        pltpu.VMEM((B, Q), jnp.float32),
        pltpu.VMEM((B, Q), jnp.int32),
    ],
)


EPW = 2 * Q
UNROLL = 4
NFULL = Q // (16 * UNROLL)


def _sc_body(boxes, scores, scale, boxes_o, bbuf, obuf, sbuf, scbuf):
    w = lax.axis_index("s") * 2 + lax.axis_index("c")
    ebase = w * EPW
    b = w // 2
    r0 = 2 * w

    pltpu.sync_copy(scale, scbuf)
    pltpu.sync_copy(boxes.at[pl.ds(ebase, EPW)], bbuf)
    pltpu.sync_copy(scores.at[pl.ds(b * Q, Q)], sbuf)

    s0 = plsc.load_gather(scbuf, [jnp.zeros((16,), jnp.int32) + r0])
    s1 = plsc.load_gather(scbuf, [jnp.zeros((16,), jnp.int32) + (r0 + 1)])

    def step(v, carry):
        base = v * (16 * UNROLL)
        for u in range(UNROLL):
            le = base + u * 16
            sc = sbuf[pl.ds(le, 16)]
            msk = sc > THRESH
            obuf[pl.ds(le, 16)] = jnp.where(msk, bbuf[pl.ds(le, 16)] * s0, 0.0)
            obuf[pl.ds(Q + le, 16)] = jnp.where(
                msk, bbuf[pl.ds(Q + le, 16)] * s1, 0.0)
        return carry

    lax.fori_loop(0, NFULL, step, 0)

    sc = sbuf[pl.ds(Q - 16, 16)]
    msk = sc > THRESH
    obuf[pl.ds(Q - 16, 16)] = jnp.where(msk, bbuf[pl.ds(Q - 16, 16)] * s0, 0.0)
    obuf[pl.ds(EPW - 16, 16)] = jnp.where(msk, bbuf[pl.ds(EPW - 16, 16)] * s1, 0.0)

    pltpu.sync_copy(obuf, boxes_o.at[pl.ds(ebase, EPW)])


_sc_boxes = pl.kernel(
    _sc_body,
    out_type=jax.ShapeDtypeStruct((N * 4,), jnp.float32),
    mesh=plsc.VectorSubcoreMesh(core_axis_name="c", subcore_axis_name="s"),
    scratch_types=[
        pltpu.VMEM((EPW,), jnp.float32),
        pltpu.VMEM((EPW,), jnp.float32),
        pltpu.VMEM((Q,), jnp.float32),
        pltpu.VMEM((64,), jnp.float32),
    ],
    compiler_params=pltpu.CompilerParams(needs_layout_passes=False),
)


@jax.jit
def kernel(pred_logits, pred_boxes, target_sizes):
    lgT = jnp.transpose(pred_logits, (2, 0, 1))
    scores2d, labels2d = _tc_scores(lgT)

    ts = target_sizes.astype(jnp.float32)
    img_h = ts[:, 0]
    img_w = ts[:, 1]
    scale = jnp.stack([img_w, img_h, img_w, img_h], axis=1).reshape(-1)

    bxt = jnp.transpose(pred_boxes, (0, 2, 1)).reshape(-1)
    boxes_t = _sc_boxes(bxt, scores2d.reshape(N), scale)
    boxes = jnp.transpose(boxes_t.reshape(B, 4, Q), (0, 2, 1))

    keep = scores2d > THRESH
    return scores2d, labels2d, boxes, keep
